# baseline (speedup 1.0000x reference)
import functools

import jax
import jax.numpy as jnp
from jax import lax
from jax.experimental import pallas as pl
from jax.experimental.pallas import tpu as pltpu
from jax.experimental.pallas import tpu_sc as plsc

VOCAB = 1000000
D = 128
ODIM = 5
B = 4096
H = 200
LANES = 16
DCH = D // LANES

CH0, CH1 = 104, 96
NBUF = 4


def _sc_info():
    try:
        info = plsc.get_sparse_core_info()
        return info.num_cores, info.num_subcores
    except Exception:
        return 2, 16


def _make_pooled_kernel():
    nc, ns = _sc_info()
    nw = nc * ns
    b_per_w = B // nw
    mesh = plsc.VectorSubcoreMesh(
        core_axis_name="c", subcore_axis_name="s",
        num_cores=nc, num_subcores=ns)

    @functools.partial(
        pl.kernel,
        out_type=jax.ShapeDtypeStruct((B, D), jnp.float32),
        mesh=mesh,
        scratch_types=[
            pltpu.VMEM((b_per_w * H,), jnp.int32),
            pltpu.VMEM((NBUF, H, D), jnp.float32),
            pltpu.VMEM((NBUF, D), jnp.float32),
            [pltpu.SemaphoreType.DMA] * NBUF,
            [pltpu.SemaphoreType.DMA] * NBUF,
            [pltpu.SemaphoreType.DMA] * NBUF,
        ],
    )
    def pooled_kernel(idx_hbm, table_hbm, out_hbm, idx_v, rows_v, orow_v,
                      sems_c0, sems_c1, sems_o):
        cid = lax.axis_index("c")
        sid = lax.axis_index("s")
        wid = sid * nc + cid
        base = wid * b_per_w

        pltpu.sync_copy(idx_hbm.at[pl.ds(base * H, b_per_w * H)], idx_v)

        def gather_copies(b, buf):
            off = b * H
            c0 = pltpu.make_async_copy(
                table_hbm.at[idx_v.at[pl.ds(off, CH0)]],
                rows_v.at[buf, pl.ds(0, CH0)],
                sems_c0[buf],
            )
            c1 = pltpu.make_async_copy(
                table_hbm.at[idx_v.at[pl.ds(off + CH0, CH1)]],
                rows_v.at[buf, pl.ds(CH0, CH1)],
                sems_c1[buf],
            )
            return c0, c1

        def issue(b, buf):
            c0, c1 = gather_copies(b, buf)
            c0.start()
            c1.start()

        for b0 in range(NBUF):
            issue(b0, b0)

        def acc_range(buf, lo, hi, accs):
            def acc_body(l, accs):
                return tuple(
                    accs[j] + rows_v[buf, l, pl.ds(LANES * j, LANES)]
                    for j in range(DCH)
                )
            return lax.fori_loop(lo, hi, acc_body, accs)

        def out_copy(b, buf):
            return pltpu.make_async_copy(
                orow_v.at[buf], out_hbm.at[base + b], sems_o[buf])

        def process(b, buf, first_round):
            c0, c1 = gather_copies(b, buf)
            accs = tuple(
                jnp.zeros((LANES,), jnp.float32) for _ in range(DCH))
            c0.wait()
            accs = acc_range(buf, 0, CH0, accs)
            c1.wait()
            accs = acc_range(buf, CH0, H, accs)
            if not first_round:
                out_copy(b - NBUF, buf).wait()
            for j in range(DCH):
                orow_v[buf, pl.ds(LANES * j, LANES)] = accs[j]
            out_copy(b, buf).start()

        def outer(i, carry):
            for buf in range(NBUF):
                b = i * NBUF + buf
                process(b, buf, first_round=False)

                @pl.when(b + NBUF < b_per_w)
                def _():
                    issue(b + NBUF, buf)
            return carry

        for buf in range(NBUF):
            process(buf, buf, first_round=True)
            issue(buf + NBUF, buf)
        lax.fori_loop(1, b_per_w // NBUF, outer, 0)
        for buf in range(NBUF):
            out_copy(b_per_w - NBUF + buf, buf).wait()

    return pooled_kernel


def _mm_body(p_ref, wt_ref, b_ref, o_ref):
    o_ref[...] = (
        jnp.dot(p_ref[...], wt_ref[...], preferred_element_type=jnp.float32)
        + jnp.float32(H) * b_ref[...]
    )


def kernel(inputs, embed_weight, linear_w, linear_b):
    idx_flat = jnp.reshape(inputs, (B * H,)).astype(jnp.int32)
    pooled = _make_pooled_kernel()(idx_flat, embed_weight)
    out = pl.pallas_call(
        _mm_body,
        out_shape=jax.ShapeDtypeStruct((B, ODIM), jnp.float32),
    )(pooled, linear_w.T, jnp.reshape(linear_b, (1, ODIM)))
    return out
